# baseline (speedup 1.0000x reference)
import functools

import jax
import jax.numpy as jnp
from jax import lax
from jax.experimental import pallas as pl
from jax.experimental.pallas import tpu as pltpu
from jax.experimental.pallas import tpu_sc as plsc

B = 16384
D = 32
NC = 2
NS = 16
NW = NC * NS
BPW = B // NW
CHUNK = 128
NCHUNK = BPW // CHUNK
L = 16


def _sc_body(uid_hbm, iid_hbm, ue_hbm, ie_hbm, ub_hbm, ib_hbm, out_hbm,
             uid_v, iid_v, ue_v, ie_v, ub_v, ib_v, out_v, sem):
  wid = lax.axis_index("s") * NC + lax.axis_index("c")
  base = pl.multiple_of(wid * BPW, BPW)

  pltpu.sync_copy(uid_hbm.at[pl.ds(base, BPW)], uid_v)
  pltpu.sync_copy(iid_hbm.at[pl.ds(base, BPW)], iid_v)

  copies = []
  for j in range(NCHUNK):
    idx_u = uid_v.at[pl.ds(j * CHUNK, CHUNK)]
    idx_i = iid_v.at[pl.ds(j * CHUNK, CHUNK)]
    copies.append(pltpu.async_copy(
        ue_hbm.at[idx_u], ue_v.at[pl.ds(j * CHUNK, CHUNK)], sem))
    copies.append(pltpu.async_copy(
        ie_hbm.at[idx_i], ie_v.at[pl.ds(j * CHUNK, CHUNK)], sem))
    copies.append(pltpu.async_copy(
        ub_hbm.at[idx_u], ub_v.at[pl.ds(j * CHUNK, CHUNK)], sem))
    copies.append(pltpu.async_copy(
        ib_hbm.at[idx_i], ib_v.at[pl.ds(j * CHUNK, CHUNK)], sem))
  for cp in copies:
    cp.wait()

  lanes = lax.iota(jnp.int32, L)

  def group(g, carry):
    base_r = pl.multiple_of(g * L, L)
    acc = ub_v[pl.ds(base_r, L)] + ib_v[pl.ds(base_r, L)]
    for k in range(L):
      u0 = ue_v[base_r + k, pl.ds(0, L)]
      v0 = ie_v[base_r + k, pl.ds(0, L)]
      u1 = ue_v[base_r + k, pl.ds(L, L)]
      v1 = ie_v[base_r + k, pl.ds(L, L)]
      s = jnp.sum(u0 * v0 + u1 * v1)
      acc = jnp.where(lanes == k, acc + s, acc)
    out_v[pl.ds(base_r, L)] = acc
    return carry

  lax.fori_loop(0, BPW // L, group, 0)

  pltpu.sync_copy(out_v, out_hbm.at[pl.ds(base, BPW)])


@jax.jit
def _run(user_ids, item_ids, user_emb, item_emb, user_bias, item_bias):
  mesh = plsc.VectorSubcoreMesh(
      core_axis_name="c", subcore_axis_name="s",
      num_cores=NC, num_subcores=NS)
  f = pl.kernel(
      _sc_body,
      out_type=jax.ShapeDtypeStruct((B,), jnp.float32),
      mesh=mesh,
      scratch_types=[
          pltpu.VMEM((BPW,), jnp.int32),
          pltpu.VMEM((BPW,), jnp.int32),
          pltpu.VMEM((BPW, D), jnp.float32),
          pltpu.VMEM((BPW, D), jnp.float32),
          pltpu.VMEM((BPW,), jnp.float32),
          pltpu.VMEM((BPW,), jnp.float32),
          pltpu.VMEM((BPW,), jnp.float32),
          pltpu.SemaphoreType.DMA,
      ],
      compiler_params=pltpu.CompilerParams(
          needs_layout_passes=False, use_tc_tiling_on_sc=False),
  )
  return f(user_ids, item_ids, user_emb, item_emb, user_bias, item_bias)


def kernel(user_ids, item_ids, user_emb, item_emb, user_bias, item_bias):
  return _run(user_ids.astype(jnp.int32), item_ids.astype(jnp.int32),
              user_emb, item_emb,
              user_bias.reshape(-1), item_bias.reshape(-1))
